# baseline (speedup 1.0000x reference)
import functools

import jax
import jax.numpy as jnp
from jax import lax
from jax.experimental import pallas as pl
from jax.experimental.pallas import tpu as pltpu, tpu_sc as plsc

_LANES = 16
_NW = 32


_CH = 64


def _gather_body(seq, hid, ids_hbm, table_hbm, out_hbm,
                 ids_v, rows_v, sg0, sg1, so0, so1):
    per_w = seq // _NW
    nchunk = per_w // _CH
    wid = lax.axis_index("s") * 2 + lax.axis_index("c")
    base = wid * per_w

    pltpu.sync_copy(ids_hbm.at[0, pl.ds(base, per_w)], ids_v)

    sg = (sg0, sg1)
    so = (so0, so1)

    def in_desc(c, b):
        return pltpu.make_async_copy(
            table_hbm.at[ids_v.at[pl.ds(c * _CH, _CH)]],
            rows_v.at[b], sg[b])

    def out_desc(c, b):
        return pltpu.make_async_copy(
            rows_v.at[b],
            out_hbm.at[0, pl.ds(base + c * _CH, _CH)], so[b])

    def steady_step(c, b):
        out_desc(c - 1, 1 - b).wait()
        in_desc(c + 1, 1 - b).start()
        in_desc(c, b).wait()
        out_desc(c, b).start()

    in_desc(0, 0).start()
    in_desc(1, 1).start()
    in_desc(0, 0).wait()
    out_desc(0, 0).start()

    def steady(k, carry):
        c = 2 * k + 1
        steady_step(c, 1)
        steady_step(c + 1, 0)
        return carry

    lax.fori_loop(0, (nchunk - 2) // 2, steady, 0)

    c_last = nchunk - 1
    in_desc(c_last, 1).wait()
    out_desc(c_last, 1).start()
    out_desc(c_last - 1, 0).wait()
    out_desc(c_last, 1).wait()


def _sc_gather(ids, table):
    seq = ids.shape[1]
    hid = table.shape[1]
    per_w = seq // _NW
    body = functools.partial(_gather_body, seq, hid)
    return pl.kernel(
        body,
        out_type=jax.ShapeDtypeStruct((1, seq, hid), jnp.float32),
        mesh=plsc.VectorSubcoreMesh(core_axis_name="c", subcore_axis_name="s"),
        compiler_params=pltpu.CompilerParams(use_tc_tiling_on_sc=True),
        scratch_types=[
            pltpu.VMEM((per_w,), jnp.int32),
            pltpu.VMEM((2, _CH, hid), jnp.float32),
            pltpu.SemaphoreType.DMA,
            pltpu.SemaphoreType.DMA,
            pltpu.SemaphoreType.DMA,
            pltpu.SemaphoreType.DMA,
        ],
    )(ids, table)


def _ln_body(htext_ref, mask_ref, vals_ref, pos_ref, gamma_ref, beta_ref,
             out_ref):
    x = htext_ref[0]
    p = pos_ref[0]
    m = mask_ref[0].astype(jnp.float32)
    v = vals_ref[0]
    scale = (m * v - m + 1.0)[:, None]
    t = x * scale + p
    mean = jnp.mean(t, axis=-1, keepdims=True)
    var = jnp.mean(jnp.square(t - mean), axis=-1, keepdims=True)
    normed = (t - mean) * lax.rsqrt(var + 1e-5)
    out_ref[0] = normed * gamma_ref[...] + beta_ref[...]


def _tc_ln(htext, mask, vals, pos, gamma, beta):
    _, seq, hid = htext.shape
    blk = 2048
    row_spec = pl.BlockSpec((1, blk, hid), lambda i: (0, i, 0))
    tok_spec = pl.BlockSpec((1, blk), lambda i: (0, i))
    vec_spec = pl.BlockSpec((hid,), lambda i: (0,))
    return pl.pallas_call(
        _ln_body,
        grid=(seq // blk,),
        in_specs=[row_spec, tok_spec, tok_spec, row_spec, vec_spec, vec_spec],
        out_specs=row_spec,
        out_shape=jax.ShapeDtypeStruct((1, seq, hid), jnp.float32),
    )(htext, mask, vals, pos, gamma, beta)


@jax.jit
def _fused(ids, mask, vals, table, pos, gamma, beta):
    htext = _sc_gather(ids, table)
    return _tc_ln(htext, mask, vals, pos, gamma, beta)


def kernel(input_ids, num_mask, num_values, word_embeddings,
           positional_encoding, ln_gamma, ln_beta):
    ids = input_ids.astype(jnp.int32)
    return _fused(ids, num_mask, num_values.astype(jnp.float32),
                  word_embeddings, positional_encoding, ln_gamma, ln_beta)


# speedup vs baseline: 1.0074x; 1.0074x over previous
import functools

import jax
import jax.numpy as jnp
from jax import lax
from jax.experimental import pallas as pl
from jax.experimental.pallas import tpu as pltpu, tpu_sc as plsc

_LANES = 16
_NW = 32


_CH = 32


def _gather_body(seq, hid, ids_hbm, table_hbm, out_hbm,
                 ids_v, rows_v, sg0, sg1, so0, so1):
    per_w = seq // _NW
    nchunk = per_w // _CH
    wid = lax.axis_index("s") * 2 + lax.axis_index("c")
    base = wid * per_w

    pltpu.sync_copy(ids_hbm.at[0, pl.ds(base, per_w)], ids_v)

    sg = (sg0, sg1)
    so = (so0, so1)

    def in_desc(c, b):
        return pltpu.make_async_copy(
            table_hbm.at[ids_v.at[pl.ds(c * _CH, _CH)]],
            rows_v.at[b], sg[b])

    def out_desc(c, b):
        return pltpu.make_async_copy(
            rows_v.at[b],
            out_hbm.at[0, pl.ds(base + c * _CH, _CH)], so[b])

    def steady_step(c, b):
        out_desc(c - 1, 1 - b).wait()
        in_desc(c + 1, 1 - b).start()
        in_desc(c, b).wait()
        out_desc(c, b).start()

    in_desc(0, 0).start()
    in_desc(1, 1).start()
    in_desc(0, 0).wait()
    out_desc(0, 0).start()

    def steady(k, carry):
        c = 2 * k + 1
        steady_step(c, 1)
        steady_step(c + 1, 0)
        return carry

    lax.fori_loop(0, (nchunk - 2) // 2, steady, 0)

    c_last = nchunk - 1
    in_desc(c_last, 1).wait()
    out_desc(c_last, 1).start()
    out_desc(c_last - 1, 0).wait()
    out_desc(c_last, 1).wait()


def _sc_gather(ids, table):
    seq = ids.shape[1]
    hid = table.shape[1]
    per_w = seq // _NW
    body = functools.partial(_gather_body, seq, hid)
    return pl.kernel(
        body,
        out_type=jax.ShapeDtypeStruct((1, seq, hid), jnp.float32),
        mesh=plsc.VectorSubcoreMesh(core_axis_name="c", subcore_axis_name="s"),
        compiler_params=pltpu.CompilerParams(use_tc_tiling_on_sc=True),
        scratch_types=[
            pltpu.VMEM((per_w,), jnp.int32),
            pltpu.VMEM((2, _CH, hid), jnp.float32),
            pltpu.SemaphoreType.DMA,
            pltpu.SemaphoreType.DMA,
            pltpu.SemaphoreType.DMA,
            pltpu.SemaphoreType.DMA,
        ],
    )(ids, table)


def _ln_body(htext_ref, mask_ref, vals_ref, pos_ref, gamma_ref, beta_ref,
             out_ref):
    x = htext_ref[0]
    p = pos_ref[0]
    m = mask_ref[0].astype(jnp.float32)
    v = vals_ref[0]
    scale = (m * v - m + 1.0)[:, None]
    t = x * scale + p
    mean = jnp.mean(t, axis=-1, keepdims=True)
    var = jnp.mean(jnp.square(t - mean), axis=-1, keepdims=True)
    normed = (t - mean) * lax.rsqrt(var + 1e-5)
    out_ref[0] = normed * gamma_ref[...] + beta_ref[...]


def _tc_ln(htext, mask, vals, pos, gamma, beta):
    _, seq, hid = htext.shape
    blk = 2048
    row_spec = pl.BlockSpec((1, blk, hid), lambda i: (0, i, 0))
    tok_spec = pl.BlockSpec((1, blk), lambda i: (0, i))
    vec_spec = pl.BlockSpec((hid,), lambda i: (0,))
    return pl.pallas_call(
        _ln_body,
        grid=(seq // blk,),
        in_specs=[row_spec, tok_spec, tok_spec, row_spec, vec_spec, vec_spec],
        out_specs=row_spec,
        out_shape=jax.ShapeDtypeStruct((1, seq, hid), jnp.float32),
    )(htext, mask, vals, pos, gamma, beta)


@jax.jit
def _fused(ids, mask, vals, table, pos, gamma, beta):
    htext = _sc_gather(ids, table)
    return _tc_ln(htext, mask, vals, pos, gamma, beta)


def kernel(input_ids, num_mask, num_values, word_embeddings,
           positional_encoding, ln_gamma, ln_beta):
    ids = input_ids.astype(jnp.int32)
    return _fused(ids, num_mask, num_values.astype(jnp.float32),
                  word_embeddings, positional_encoding, ln_gamma, ln_beta)
